# baseline (speedup 1.0000x reference)
import functools

import jax
import jax.numpy as jnp
from jax import lax
from jax.experimental import pallas as pl
from jax.experimental.pallas import tpu as pltpu
from jax.experimental.pallas import tpu_sc as plsc

B = 16384
D = 64
NC = 2
NS = 16
L = 16
NW = NC * NS
RPW = B // NW

_mesh = plsc.VectorSubcoreMesh(core_axis_name="c", subcore_axis_name="s")


@functools.partial(
    pl.kernel,
    mesh=_mesh,
    compiler_params=pltpu.CompilerParams(needs_layout_passes=False),
    out_type=jax.ShapeDtypeStruct((B,), jnp.float32),
    scratch_types=[
        pltpu.VMEM((RPW * D,), jnp.float32),
        pltpu.VMEM((RPW * D,), jnp.float32),
        pltpu.VMEM((RPW,), jnp.float32),
        pltpu.VMEM((RPW,), jnp.float32),
        pltpu.VMEM((L,), jnp.float32),
        pltpu.VMEM((RPW,), jnp.float32),
        pltpu.VMEM((RPW,), jnp.float32),
    ],
)
def _sc_kernel(gu_hbm, gi_hbm, bu_hbm, bi_hbm, mu_hbm, out_hbm,
               gu_v, gi_v, bu_v, bi_v, mu_v, tmp_v, out_v):
    c = lax.axis_index("c")
    s = lax.axis_index("s")
    wid = s * NC + c
    base = wid * RPW
    pltpu.sync_copy(gu_hbm.at[pl.ds(base * D, RPW * D)], gu_v)
    pltpu.sync_copy(gi_hbm.at[pl.ds(base * D, RPW * D)], gi_v)
    pltpu.sync_copy(bu_hbm.at[pl.ds(base, RPW)], bu_v)
    pltpu.sync_copy(bi_hbm.at[pl.ds(base, RPW)], bi_v)
    pltpu.sync_copy(mu_hbm, mu_v)
    mu_vec = mu_v[...]
    lanes = lax.iota(jnp.int32, L)

    def grp(g, carry):
        r0 = g * L
        outv = mu_vec
        for u in range(L):
            off = (r0 + u) * D
            acc = (gu_v[pl.ds(off, L)] * gi_v[pl.ds(off, L)]
                   + gu_v[pl.ds(off + L, L)] * gi_v[pl.ds(off + L, L)])
            acc = acc + (gu_v[pl.ds(off + 2 * L, L)] * gi_v[pl.ds(off + 2 * L, L)]
                         + gu_v[pl.ds(off + 3 * L, L)] * gi_v[pl.ds(off + 3 * L, L)])
            outv = jnp.where(lanes == u, jnp.sum(acc), outv)
        sl = pl.ds(r0, L)
        out_v[sl] = outv + bu_v[sl] + bi_v[sl] + mu_vec
        return carry

    lax.fori_loop(0, RPW // L, grp, 0)
    pltpu.sync_copy(out_v, out_hbm.at[pl.ds(base, RPW)])


def kernel(gu, gi, bu, bi, Mu):
    guf = gu.reshape(B * D)
    gif = gi.reshape(B * D)
    bu1 = bu.reshape(B)
    bi1 = bi.reshape(B)
    mu16 = jnp.broadcast_to(Mu.reshape(()), (L,))
    return _sc_kernel(guf, gif, bu1, bi1, mu16)


# speedup vs baseline: 1.2743x; 1.2743x over previous
import functools

import jax
import jax.numpy as jnp
from jax import lax
from jax.experimental import pallas as pl
from jax.experimental.pallas import tpu as pltpu
from jax.experimental.pallas import tpu_sc as plsc

B = 16384
D = 64
NC = 2
NS = 16
L = 16
NW = NC * NS
RPW = B // NW
CH = 128
NCHK = RPW // CH

_mesh = plsc.VectorSubcoreMesh(core_axis_name="c", subcore_axis_name="s")


@functools.partial(
    pl.kernel,
    mesh=_mesh,
    compiler_params=pltpu.CompilerParams(needs_layout_passes=False),
    out_type=jax.ShapeDtypeStruct((B,), jnp.float32),
    scratch_types=[
        pltpu.VMEM((CH, D), jnp.float32),
        pltpu.VMEM((CH, D), jnp.float32),
        pltpu.VMEM((CH, D), jnp.float32),
        pltpu.VMEM((CH, D), jnp.float32),
        pltpu.VMEM((RPW,), jnp.float32),
        pltpu.VMEM((RPW,), jnp.float32),
        pltpu.VMEM((L,), jnp.float32),
        pltpu.VMEM((RPW,), jnp.float32),
        pltpu.SemaphoreType.DMA,
        pltpu.SemaphoreType.DMA,
        pltpu.SemaphoreType.DMA,
        pltpu.SemaphoreType.DMA,
    ],
)
def _sc_kernel(gu_hbm, gi_hbm, bu_hbm, bi_hbm, mu_hbm, out_hbm,
               gu_a, gu_b, gi_a, gi_b, bu_v, bi_v, mu_v, out_v,
               s0, s1, s2, s3):
    c = lax.axis_index("c")
    s = lax.axis_index("s")
    wid = s * NC + c
    base = wid * RPW
    pltpu.sync_copy(mu_hbm, mu_v)
    pltpu.sync_copy(bu_hbm.at[pl.ds(base, RPW)], bu_v)
    pltpu.sync_copy(bi_hbm.at[pl.ds(base, RPW)], bi_v)
    mu_vec = mu_v[...]
    lanes = lax.iota(jnp.int32, L)

    gub = [gu_a, gu_b]
    gib = [gi_a, gi_b]
    sems = [s0, s1, s2, s3]

    def start(k):
        buf = k % 2
        r0 = base + k * CH
        return (pltpu.async_copy(gu_hbm.at[pl.ds(r0, CH), :], gub[buf],
                                 sems[2 * buf]),
                pltpu.async_copy(gi_hbm.at[pl.ds(r0, CH), :], gib[buf],
                                 sems[2 * buf + 1]))

    pend = {0: start(0)}
    for k in range(NCHK):
        if k + 1 < NCHK:
            pend[k + 1] = start(k + 1)
        for h in pend.pop(k):
            h.wait()
        guv = gub[k % 2]
        giv = gib[k % 2]

        def grp(g, carry):
            r0 = g * L
            outv = mu_vec
            for u in range(L):
                r = r0 + u
                acc = (guv[r, pl.ds(0, L)] * giv[r, pl.ds(0, L)]
                       + guv[r, pl.ds(L, L)] * giv[r, pl.ds(L, L)])
                acc = acc + (guv[r, pl.ds(2 * L, L)] * giv[r, pl.ds(2 * L, L)]
                             + guv[r, pl.ds(3 * L, L)] * giv[r, pl.ds(3 * L, L)])
                outv = jnp.where(lanes == u, jnp.sum(acc), outv)
            sl = pl.ds(k * CH + g * L, L)
            out_v[sl] = outv + bu_v[sl] + bi_v[sl] + mu_vec
            return carry

        lax.fori_loop(0, CH // L, grp, 0)

    pltpu.sync_copy(out_v, out_hbm.at[pl.ds(base, RPW)])


def kernel(gu, gi, bu, bi, Mu):
    mu16 = jnp.broadcast_to(Mu.reshape(()), (L,))
    return _sc_kernel(gu, gi, bu.reshape(B), bi.reshape(B), mu16)


# speedup vs baseline: 1.2782x; 1.0031x over previous
import functools

import jax
import jax.numpy as jnp
from jax import lax
from jax.experimental import pallas as pl
from jax.experimental.pallas import tpu as pltpu
from jax.experimental.pallas import tpu_sc as plsc

B = 16384
D = 64
NC = 2
NS = 16
L = 16
NW = NC * NS
RPW = B // NW
CH = 128
NCHK = RPW // CH

_mesh = plsc.VectorSubcoreMesh(core_axis_name="c", subcore_axis_name="s")


@functools.partial(
    pl.kernel,
    mesh=_mesh,
    compiler_params=pltpu.CompilerParams(needs_layout_passes=False,
                                         use_tc_tiling_on_sc=True),
    out_type=jax.ShapeDtypeStruct((B,), jnp.float32),
    scratch_types=[
        pltpu.VMEM((CH, D), jnp.float32),
        pltpu.VMEM((CH, D), jnp.float32),
        pltpu.VMEM((CH, D), jnp.float32),
        pltpu.VMEM((CH, D), jnp.float32),
        pltpu.VMEM((RPW,), jnp.float32),
        pltpu.VMEM((RPW,), jnp.float32),
        pltpu.VMEM((L,), jnp.float32),
        pltpu.VMEM((RPW,), jnp.float32),
        pltpu.SemaphoreType.DMA,
        pltpu.SemaphoreType.DMA,
        pltpu.SemaphoreType.DMA,
        pltpu.SemaphoreType.DMA,
    ],
)
def _sc_kernel(gu_hbm, gi_hbm, bu_hbm, bi_hbm, mu_hbm, out_hbm,
               gu_a, gu_b, gi_a, gi_b, bu_v, bi_v, mu_v, out_v,
               s0, s1, s2, s3):
    c = lax.axis_index("c")
    s = lax.axis_index("s")
    wid = s * NC + c
    base = wid * RPW
    pltpu.sync_copy(mu_hbm, mu_v)
    pltpu.sync_copy(bu_hbm.at[pl.ds(base, RPW)], bu_v)
    pltpu.sync_copy(bi_hbm.at[pl.ds(base, RPW)], bi_v)
    mu_vec = mu_v[...]
    lanes = lax.iota(jnp.int32, L)

    gub = [gu_a, gu_b]
    gib = [gi_a, gi_b]
    sems = [s0, s1, s2, s3]

    def start(k):
        buf = k % 2
        r0 = base + k * CH
        return (pltpu.async_copy(gu_hbm.at[pl.ds(r0, CH), :], gub[buf],
                                 sems[2 * buf]),
                pltpu.async_copy(gi_hbm.at[pl.ds(r0, CH), :], gib[buf],
                                 sems[2 * buf + 1]))

    pend = {0: start(0)}
    for k in range(NCHK):
        if k + 1 < NCHK:
            pend[k + 1] = start(k + 1)
        for h in pend.pop(k):
            h.wait()
        guv = gub[k % 2]
        giv = gib[k % 2]

        def grp(g, carry):
            r0 = g * L
            outv = mu_vec
            for u in range(L):
                r = r0 + u
                acc = (guv[r, pl.ds(0, L)] * giv[r, pl.ds(0, L)]
                       + guv[r, pl.ds(L, L)] * giv[r, pl.ds(L, L)])
                acc = acc + (guv[r, pl.ds(2 * L, L)] * giv[r, pl.ds(2 * L, L)]
                             + guv[r, pl.ds(3 * L, L)] * giv[r, pl.ds(3 * L, L)])
                outv = jnp.where(lanes == u, jnp.sum(acc), outv)
            sl = pl.ds(k * CH + g * L, L)
            out_v[sl] = outv + bu_v[sl] + bi_v[sl] + mu_vec
            return carry

        lax.fori_loop(0, CH // L, grp, 0)

    pltpu.sync_copy(out_v, out_hbm.at[pl.ds(base, RPW)])


def kernel(gu, gi, bu, bi, Mu):
    mu16 = jnp.broadcast_to(Mu.reshape(()), (L,))
    return _sc_kernel(gu, gi, bu.reshape(B), bi.reshape(B), mu16)


# speedup vs baseline: 1.7648x; 1.3807x over previous
import functools

import jax
import jax.numpy as jnp
from jax import lax
from jax.experimental import pallas as pl
from jax.experimental.pallas import tpu as pltpu
from jax.experimental.pallas import tpu_sc as plsc

B = 16384
D = 64
NC = 2
NS = 16
L = 16
NW = NC * NS
RPW = B // NW
CPW = RPW // 2

_mesh = plsc.VectorSubcoreMesh(core_axis_name="c", subcore_axis_name="s")


@functools.partial(
    pl.kernel,
    mesh=_mesh,
    compiler_params=pltpu.CompilerParams(needs_layout_passes=False),
    out_type=jax.ShapeDtypeStruct((B,), jnp.float32),
    scratch_types=[
        pltpu.VMEM((D, CPW), jnp.float32),
        pltpu.VMEM((D, CPW), jnp.float32),
        pltpu.VMEM((D, CPW), jnp.float32),
        pltpu.VMEM((D, CPW), jnp.float32),
        pltpu.VMEM((1, RPW), jnp.float32),
        pltpu.VMEM((1, RPW), jnp.float32),
        pltpu.VMEM((L,), jnp.float32),
        pltpu.VMEM((RPW,), jnp.float32),
        pltpu.SemaphoreType.DMA,
        pltpu.SemaphoreType.DMA,
        pltpu.SemaphoreType.DMA,
        pltpu.SemaphoreType.DMA,
    ],
)
def _sc_kernel(guT_hbm, giT_hbm, buT_hbm, biT_hbm, mu_hbm, out_hbm,
               gu_a, gu_b, gi_a, gi_b, bu_v, bi_v, mu_v, out_v,
               s0, s1, s2, s3):
    c = lax.axis_index("c")
    s = lax.axis_index("s")
    wid = s * NC + c
    base = wid * RPW
    pltpu.sync_copy(mu_hbm, mu_v)
    pltpu.sync_copy(buT_hbm.at[:, pl.ds(base, RPW)], bu_v)
    pltpu.sync_copy(biT_hbm.at[:, pl.ds(base, RPW)], bi_v)
    mu_vec = mu_v[...]

    gub = [gu_a, gu_b]
    gib = [gi_a, gi_b]
    sems = [s0, s1, s2, s3]

    def start(h):
        c0 = base + h * CPW
        return (pltpu.async_copy(guT_hbm.at[:, pl.ds(c0, CPW)], gub[h],
                                 sems[2 * h]),
                pltpu.async_copy(giT_hbm.at[:, pl.ds(c0, CPW)], gib[h],
                                 sems[2 * h + 1]))

    pend = {0: start(0)}
    for h in range(2):
        if h + 1 < 2:
            pend[h + 1] = start(h + 1)
        for hd in pend.pop(h):
            hd.wait()
        guv = gub[h]
        giv = gib[h]

        def grp(g, carry):
            j0 = g * L
            sl = pl.ds(j0, L)
            a0 = guv[0, sl] * giv[0, sl]
            a1 = guv[1, sl] * giv[1, sl]
            a2 = guv[2, sl] * giv[2, sl]
            a3 = guv[3, sl] * giv[3, sl]
            for d in range(4, D, 4):
                a0 = a0 + guv[d, sl] * giv[d, sl]
                a1 = a1 + guv[d + 1, sl] * giv[d + 1, sl]
                a2 = a2 + guv[d + 2, sl] * giv[d + 2, sl]
                a3 = a3 + guv[d + 3, sl] * giv[d + 3, sl]
            jr = h * CPW + j0
            slr = pl.ds(jr, L)
            out_v[slr] = ((a0 + a1) + (a2 + a3)
                          + bu_v[0, slr] + bi_v[0, slr] + mu_vec)
            return carry

        lax.fori_loop(0, CPW // L, grp, 0)

    pltpu.sync_copy(out_v, out_hbm.at[pl.ds(base, RPW)])


def kernel(gu, gi, bu, bi, Mu):
    mu16 = jnp.broadcast_to(Mu.reshape(()), (L,))
    return _sc_kernel(gu.T, gi.T, bu.T, bi.T, mu16)
